# baseline (speedup 1.0000x reference)
import functools

import jax
import jax.numpy as jnp
from jax import lax
from jax.experimental import pallas as pl
from jax.experimental.pallas import tpu as pltpu
from jax.experimental.pallas import tpu_sc as plsc

BATCH = 4096
HIST = 200
EMBED_DIM = 32

_NC = 2
_NS = 16
_NW = _NC * _NS
_CPW = BATCH // _NW
_NBUF = 4
_NGRP = HIST // _NBUF
_L = 16

_mesh = plsc.VectorSubcoreMesh(core_axis_name="c", subcore_axis_name="s")


@functools.partial(
    pl.kernel,
    mesh=_mesh,
    out_type=jax.ShapeDtypeStruct(
        (HIST, EMBED_DIM // 8, 32, 8, _CPW), jnp.float32),
    scratch_types=[
        pltpu.VMEM((HIST, _CPW), jnp.int32),
        pltpu.VMEM((_NBUF, _CPW, EMBED_DIM), jnp.float32),
        pltpu.VMEM((_NBUF, EMBED_DIM // 8, 8, _CPW), jnp.float32),
        [pltpu.SemaphoreType.DMA] * _NBUF,
        [pltpu.SemaphoreType.DMA] * _NBUF,
    ],
    compiler_params=pltpu.CompilerParams(
        use_tc_tiling_on_sc=False, needs_layout_passes=False),
)
def _embed_sc(xt_hbm, w_hbm, out_hbm, idx_v, rows_v, tile_v, gsems, ssems):
    wid = lax.axis_index("s") * _NC + lax.axis_index("c")
    c0 = pl.multiple_of(wid * _CPW, _CPW)

    pltpu.sync_copy(xt_hbm.at[:, pl.ds(c0, _CPW)], idx_v)

    lane = lax.iota(jnp.int32, _L)
    row_ids = [lane + g * _L for g in range(_CPW // _L)]

    def group(j, carry):
        h0 = j * _NBUF
        gathers = []
        for b in range(_NBUF):
            gathers.append(pltpu.async_copy(
                w_hbm.at[idx_v.at[h0 + b]], rows_v.at[b], gsems[b]))
        stores = []
        for b in range(_NBUF):
            gathers[b].wait()
            rows = rows_v.at[b]
            for d in range(EMBED_DIM):
                col = jnp.full((_L,), d, jnp.int32)
                for g in range(_CPW // _L):
                    vals = plsc.load_gather(rows, [row_ids[g], col])
                    tile_v[b, d // 8, d % 8, pl.ds(g * _L, _L)] = vals
            stores.append(pltpu.async_copy(
                tile_v.at[b], out_hbm.at[h0 + b, :, wid], ssems[b]))
        for b in range(_NBUF):
            stores[b].wait()
        return carry

    lax.fori_loop(0, _NGRP, group, 0)


def kernel(x, W):
    outp = _embed_sc(x.T, W)
    return jnp.transpose(outp, (2, 4, 0, 1, 3)).reshape(BATCH, HIST, EMBED_DIM)


# speedup vs baseline: 1.3020x; 1.3020x over previous
import functools

import jax
import jax.numpy as jnp
from jax import lax
from jax.experimental import pallas as pl
from jax.experimental.pallas import tpu as pltpu
from jax.experimental.pallas import tpu_sc as plsc

BATCH = 4096
HIST = 200
EMBED_DIM = 32

_NC = 2
_NS = 16
_NW = _NC * _NS
_CPW = BATCH // _NW
_NBUF = 8
_TH = HIST // _NBUF

_mesh = plsc.VectorSubcoreMesh(core_axis_name="c", subcore_axis_name="s")


@functools.partial(
    pl.kernel,
    mesh=_mesh,
    out_type=jax.ShapeDtypeStruct((BATCH, HIST, EMBED_DIM), jnp.float32),
    scratch_types=[
        pltpu.VMEM((_TH, _NBUF, _CPW), jnp.int32),
        pltpu.VMEM((_NBUF, _CPW, EMBED_DIM), jnp.float32),
        [pltpu.SemaphoreType.DMA] * _NBUF,
        [pltpu.SemaphoreType.DMA] * _NBUF,
    ],
    compiler_params=pltpu.CompilerParams(
        use_tc_tiling_on_sc=False, needs_layout_passes=False),
)
def _embed_sc(xp_hbm, w_hbm, out_hbm, idx_v, rows_v, gsems, ssems):
    wid = lax.axis_index("s") * _NC + lax.axis_index("c")
    c0 = pl.multiple_of(wid * _CPW, _CPW)

    pltpu.sync_copy(xp_hbm.at[:, wid], idx_v)

    def group(th, carry):
        gathers = []
        for b in range(_NBUF):
            gathers.append(pltpu.async_copy(
                w_hbm.at[idx_v.at[th, b]], rows_v.at[b], gsems[b]))
        stores = []
        for b in range(_NBUF):
            gathers[b].wait()
            stores.append(pltpu.async_copy(
                rows_v.at[b], out_hbm.at[pl.ds(c0, _CPW), th * _NBUF + b],
                ssems[b]))
        for s in stores:
            s.wait()
        return carry

    lax.fori_loop(0, _TH, group, 0)


def kernel(x, W):
    xp = x.reshape(32, 128, 25, 8).transpose(2, 0, 3, 1)
    return _embed_sc(xp, W)


# speedup vs baseline: 1.3687x; 1.0512x over previous
import functools

import jax
import jax.numpy as jnp
from jax import lax
from jax.experimental import pallas as pl
from jax.experimental.pallas import tpu as pltpu
from jax.experimental.pallas import tpu_sc as plsc

BATCH = 4096
HIST = 200
EMBED_DIM = 32

_NC = 2
_NS = 16
_NW = _NC * _NS
_CPW = BATCH // _NW
_NBUF = 8
_TH = HIST // _NBUF

_mesh = plsc.VectorSubcoreMesh(core_axis_name="c", subcore_axis_name="s")


@functools.partial(
    pl.kernel,
    mesh=_mesh,
    out_type=jax.ShapeDtypeStruct((HIST, BATCH, EMBED_DIM), jnp.float32),
    scratch_types=[
        pltpu.VMEM((_TH, _NBUF, _CPW), jnp.int32),
        pltpu.VMEM((_NBUF, _CPW, EMBED_DIM), jnp.float32),
        [pltpu.SemaphoreType.DMA] * _NBUF,
        [pltpu.SemaphoreType.DMA] * _NBUF,
    ],
    compiler_params=pltpu.CompilerParams(
        use_tc_tiling_on_sc=False, needs_layout_passes=False),
)
def _embed_sc(xp_hbm, w_hbm, out_hbm, idx_v, rows_v, gsems, ssems):
    wid = lax.axis_index("s") * _NC + lax.axis_index("c")
    c0 = pl.multiple_of(wid * _CPW, _CPW)

    pltpu.sync_copy(xp_hbm.at[:, wid], idx_v)

    def group(th, carry):
        gathers = []
        for b in range(_NBUF):
            gathers.append(pltpu.async_copy(
                w_hbm.at[idx_v.at[th, b]], rows_v.at[b], gsems[b]))
        stores = []
        for b in range(_NBUF):
            gathers[b].wait()
            stores.append(pltpu.async_copy(
                rows_v.at[b], out_hbm.at[th * _NBUF + b, pl.ds(c0, _CPW)],
                ssems[b]))
        for s in stores:
            s.wait()
        return carry

    lax.fori_loop(0, _TH, group, 0)


def kernel(x, W):
    xp = x.reshape(32, 128, 25, 8).transpose(2, 0, 3, 1)
    return jnp.swapaxes(_embed_sc(xp, W), 0, 1)
